# baseline (speedup 1.0000x reference)
import functools

import jax
import jax.numpy as jnp
from jax import lax
from jax.experimental import pallas as pl
from jax.experimental.pallas import tpu as pltpu
from jax.experimental.pallas import tpu_sc as plsc


def _conv(x, w, stride=1, pad=0):
    return lax.conv_general_dilated(
        x, w, (stride, stride), [(pad, pad), (pad, pad)],
        dimension_numbers=('NCHW', 'OIHW', 'NCHW'))


def _bn(x, p):
    return (x - p['m'][None, :, None, None]) / jnp.sqrt(
        p['v'][None, :, None, None] + 1e-5) * p['g'][None, :, None, None] \
        + p['b'][None, :, None, None]


def _bottleneck(x, blk, s):
    out = jax.nn.relu(_bn(_conv(x, blk['w1']), blk['bn1']))
    out = jax.nn.relu(_bn(_conv(out, blk['w2'], s, 1), blk['bn2']))
    out = _bn(_conv(out, blk['w3']), blk['bn3'])
    out = out + (jnp.asarray(blk['stride']) - s).astype(out.dtype)
    if 'wd' in blk:
        idn = _bn(_conv(x, blk['wd'], s), blk['bnd'])
    else:
        idn = x
    return jax.nn.relu(out + idn)


def _encode(x, enc):
    x = _conv(x, enc['conv1'], 2, 3)
    x = jax.nn.relu(_bn(x, enc['bn1']))
    x = lax.reduce_window(x, -jnp.inf, lax.max, (1, 1, 3, 3), (1, 1, 2, 2),
                          [(0, 0), (0, 0), (1, 1), (1, 1)])
    for blk in enc['layer1']:
        x = _bottleneck(x, blk, 1)
    for i, blk in enumerate(enc['layer2']):
        x = _bottleneck(x, blk, 2 if i == 0 else 1)
    for i, blk in enumerate(enc['layer3']):
        x = _bottleneck(x, blk, 2 if i == 0 else 1)
    return jnp.mean(x, axis=(2, 3))


def _dot_t(x, w):
    return lax.dot_general(x, w, (((1,), (1,)), ((), ())),
                           preferred_element_type=jnp.float32)


def _vq_sc(emb, cv):
    B, D = emb.shape
    L = plsc.get_sparse_core_info().num_lanes
    NC = 1
    DW = D
    mesh = plsc.VectorSubcoreMesh(core_axis_name="c", subcore_axis_name="s",
                                  num_cores=NC)

    @functools.partial(
        pl.kernel, mesh=mesh,
        out_type=jax.ShapeDtypeStruct((B, D), jnp.float32),
        scratch_types=[
            pltpu.VMEM((1, DW), jnp.float32),
            pltpu.VMEM((4, DW), jnp.float32),
            pltpu.VMEM((1, DW), jnp.float32),
        ],
    )
    def vq(emb_hbm, cv_hbm, mem_hbm, e_v, c_v, m_v):
        wid = lax.axis_index("s") * NC + lax.axis_index("c")
        row = wid
        col = 0
        pltpu.sync_copy(emb_hbm.at[pl.ds(row, 1), pl.ds(col, DW)], e_v)
        pltpu.sync_copy(cv_hbm.at[:, pl.ds(col, DW)], c_v)
        swp = (lax.iota(jnp.int32, L) ^ 1).reshape(L, 1)
        gd = lax.GatherDimensionNumbers(
            offset_dims=(), collapsed_slice_dims=(0,), start_index_map=(0,))

        def _swap(q):
            return lax.gather(q, swp, gd, (1,),
                              mode=lax.GatherScatterMode.PROMISE_IN_BOUNDS)

        for j in range(DW // L):
            hs = pl.ds(j * L, L)
            z = e_v[0, hs]
            r0 = z - c_v[0, hs]
            q0 = r0 * r0
            d0 = q0 + _swap(q0)
            r1 = z - c_v[1, hs]
            q1 = r1 * r1
            d1 = q1 + _swap(q1)
            pickx = d1 < d0
            m_v[0, hs] = jnp.where(pickx, c_v[3, hs], c_v[2, hs])
        pltpu.sync_copy(m_v, mem_hbm.at[pl.ds(row, 1), pl.ds(col, DW)])

    return vq(emb, cv)


def _decoder_body(m_ref, w1_ref, b1_ref, w2_ref, b2_ref, w3_ref,
                  b3_ref, out_ref):
    h = _dot_t(m_ref[...], w1_ref[...]) + b1_ref[...]
    h = _dot_t(h, w2_ref[...]) + b2_ref[...]
    h = _dot_t(h, w3_ref[...]) + b3_ref[...]
    h = h - jnp.max(h, axis=1, keepdims=True)
    eh = jnp.exp(h)
    out_ref[...] = eh / jnp.sum(eh, axis=1, keepdims=True)


def _dkvb_sc(emb, codebooks, values, W1, b1, W2, b2, W3, b3):
    B, D = emb.shape
    H = codebooks.shape[0]
    C = W3.shape[0]
    CP = 128
    cv = jnp.concatenate([codebooks.transpose(1, 0, 2).reshape(2, D),
                          values.transpose(1, 0, 2).reshape(2, D)])
    m = _vq_sc(emb, cv)
    w3 = jnp.zeros((CP, W3.shape[1]), W3.dtype).at[:C, :].set(W3)
    b3p = jnp.full((CP,), -1e30, b3.dtype).at[:C].set(b3)
    out = pl.pallas_call(
        _decoder_body,
        out_shape=jax.ShapeDtypeStruct((B, CP), jnp.float32),
    )(m, W1, b1.reshape(1, -1), W2, b2.reshape(1, -1), w3,
      b3p.reshape(1, -1))
    return out[:, :C]


def kernel(input, enc, codebooks, values, W1, b1, W2, b2, W3, b3):
    emb = lax.stop_gradient(_encode(input, enc))
    return _dkvb_sc(emb, codebooks, values, W1, b1, W2, b2, W3, b3)
